# baseline (speedup 1.0000x reference)
import functools

import jax
import jax.numpy as jnp
from jax import lax
from jax.experimental import pallas as pl
from jax.experimental.pallas import tpu as pltpu
from jax.experimental.pallas import tpu_sc as plsc

EPS_ = 1e-10
TILE_R = 128


def _loss_kernel(row_ref, col_ref, sb_ref, rm_ref, out_ref, num_acc, den_acc,
                 *, h, w, nb, nt):
    b = pl.program_id(0)
    t = pl.program_id(1)

    @pl.when(jnp.logical_and(b == 0, t == 0))
    def _init():
        num_acc[0] = 0.0
        den_acc[0] = 0.0

    x = rm_ref[0]
    row = row_ref[0, 0]
    col = col_ref[0, 0]
    sb = sb_ref[0, 0]

    iota_w = lax.broadcasted_iota(jnp.int32, (TILE_R, 1, w), 2)
    mask_w = iota_w == col[:, None, None]
    iota_h = lax.broadcasted_iota(jnp.int32, (TILE_R, h), 1)
    mask_h = iota_h == row[:, None]

    sum_w = jnp.sum(x, axis=2)
    denom = jnp.sum(sum_w, axis=1)

    srm_w = jnp.sum(jnp.where(mask_w, x, 0.0), axis=2)
    srm = jnp.sum(jnp.where(mask_h, srm_w, 0.0), axis=1)

    num_acc[0] += jnp.sum(sb * -jnp.log(EPS_ + srm / denom))
    den_acc[0] += jnp.sum(sb)

    @pl.when(jnp.logical_and(b == nb - 1, t == nt - 1))
    def _fin():
        out_ref[...] = jnp.full((1, 1), num_acc[0] / (1.0 + den_acc[0]), jnp.float32)


def _make_sc_gather(n, nc, ns, s_per_b, hw):
    nw = nc * ns
    per_w = n // nw
    mesh = plsc.VectorSubcoreMesh(core_axis_name="c", subcore_axis_name="s")

    @functools.partial(
        pl.kernel, mesh=mesh,
        out_type=jax.ShapeDtypeStruct((n,), jnp.float32),
        scratch_types=[
            pltpu.VMEM((per_w,), jnp.int32),
            pltpu.VMEM((per_w,), jnp.float32),
            pltpu.SemaphoreType.DMA,
        ],
    )
    def sc_gather(table_hbm, loc_hbm, out_hbm, idx_v, vals_v, sem):
        wid = lax.axis_index("s") * nc + lax.axis_index("c")
        base = wid * per_w
        pltpu.sync_copy(loc_hbm.at[pl.ds(base, per_w)], idx_v)
        off = (base // s_per_b) * hw
        for i in range(per_w // 16):
            sl = pl.ds(i * 16, 16)
            idx_v[sl] = idx_v[sl] + off
        pltpu.async_copy(table_hbm.at[idx_v], vals_v, sem).wait()
        pltpu.sync_copy(vals_v, out_hbm.at[pl.ds(base, per_w)])

    return sc_gather


def kernel(response_map, source_feature_1d_locations, boundaries):
    B, S, H, W = response_map.shape
    HW = H * W
    T = S // TILE_R
    NG = B * T

    loc = source_feature_1d_locations.astype(jnp.int32)
    row = (loc // W).reshape(NG, 1, TILE_R)
    col = (loc % W).reshape(NG, 1, TILE_R)
    loc_flat = loc.reshape(B * S)

    bnd_flat = boundaries.reshape(B * HW)

    info = plsc.get_sparse_core_info()
    sb = _make_sc_gather(B * S, info.num_cores, info.num_subcores, S, HW)(
        bnd_flat, loc_flat)
    sb3 = sb.reshape(NG, 1, TILE_R)

    out = pl.pallas_call(
        functools.partial(_loss_kernel, h=H, w=W, nb=B, nt=T),
        grid=(B, T),
        in_specs=[
            pl.BlockSpec((1, 1, TILE_R), lambda b, t: (b * T + t, 0, 0)),
            pl.BlockSpec((1, 1, TILE_R), lambda b, t: (b * T + t, 0, 0)),
            pl.BlockSpec((1, 1, TILE_R), lambda b, t: (b * T + t, 0, 0)),
            pl.BlockSpec((1, TILE_R, H, W), lambda b, t: (b, t, 0, 0)),
        ],
        out_specs=pl.BlockSpec((1, 1), lambda b, t: (0, 0)),
        out_shape=jax.ShapeDtypeStruct((1, 1), jnp.float32),
        scratch_shapes=[
            pltpu.SMEM((1,), jnp.float32),
            pltpu.SMEM((1,), jnp.float32),
        ],
    )(row, col, sb3, response_map)
    return out[0, 0]


# speedup vs baseline: 1.0040x; 1.0040x over previous
import functools

import jax
import jax.numpy as jnp
from jax import lax
from jax.experimental import pallas as pl
from jax.experimental.pallas import tpu as pltpu
from jax.experimental.pallas import tpu_sc as plsc

EPS_ = 1e-10
TILE_R = 128


def _loss_kernel(row_ref, col_ref, sb_ref, rm_ref, out_ref, num_acc, den_acc,
                 *, h, w, nb, nt):
    b = pl.program_id(0)
    t = pl.program_id(1)

    @pl.when(jnp.logical_and(b == 0, t == 0))
    def _init():
        num_acc[0] = 0.0
        den_acc[0] = 0.0

    x = rm_ref[0]
    row = row_ref[0, 0]
    col = col_ref[0, 0]
    sb = sb_ref[0, 0]

    iota_w = lax.broadcasted_iota(jnp.int32, (TILE_R, 1, w), 2)
    mask_w = iota_w == col[:, None, None]
    iota_h = lax.broadcasted_iota(jnp.int32, (TILE_R, h), 1)
    mask_h = iota_h == row[:, None]

    sum_w = jnp.sum(x, axis=2)
    denom = jnp.sum(sum_w, axis=1)

    srm_w = jnp.sum(jnp.where(mask_w, x, 0.0), axis=2)
    srm = jnp.sum(jnp.where(mask_h, srm_w, 0.0), axis=1)

    num_acc[0] += jnp.sum(sb * -jnp.log(EPS_ + srm / denom))
    den_acc[0] += jnp.sum(sb)

    @pl.when(jnp.logical_and(b == nb - 1, t == nt - 1))
    def _fin():
        out_ref[...] = jnp.full((1, 1), num_acc[0] / (1.0 + den_acc[0]), jnp.float32)


def _make_sc_gather(n, nc, ns):
    nw = nc * ns
    per_w = n // nw
    mesh = plsc.VectorSubcoreMesh(core_axis_name="c", subcore_axis_name="s")

    @functools.partial(
        pl.kernel, mesh=mesh,
        out_type=jax.ShapeDtypeStruct((n,), jnp.float32),
        scratch_types=[
            pltpu.VMEM((per_w,), jnp.int32),
            pltpu.VMEM((per_w,), jnp.float32),
            pltpu.SemaphoreType.DMA,
        ],
    )
    def sc_gather(table_hbm, idx_hbm, out_hbm, idx_v, vals_v, sem):
        wid = lax.axis_index("s") * nc + lax.axis_index("c")
        base = wid * per_w
        pltpu.sync_copy(idx_hbm.at[pl.ds(base, per_w)], idx_v)
        pltpu.async_copy(table_hbm.at[idx_v], vals_v, sem).wait()
        pltpu.sync_copy(vals_v, out_hbm.at[pl.ds(base, per_w)])

    return sc_gather


def kernel(response_map, source_feature_1d_locations, boundaries):
    B, S, H, W = response_map.shape
    HW = H * W
    T = S // TILE_R
    NG = B * T

    loc = source_feature_1d_locations.astype(jnp.int32)
    row = (loc // W).reshape(NG, 1, TILE_R)
    col = (loc % W).reshape(NG, 1, TILE_R)
    gidx = (jnp.arange(B, dtype=jnp.int32)[:, None] * HW + loc).reshape(B * S)

    bnd_flat = boundaries.reshape(B * HW)

    info = plsc.get_sparse_core_info()
    sb = _make_sc_gather(B * S, info.num_cores, info.num_subcores)(bnd_flat, gidx)
    sb3 = sb.reshape(NG, 1, TILE_R)

    out = pl.pallas_call(
        functools.partial(_loss_kernel, h=H, w=W, nb=B, nt=T),
        grid=(B, T),
        in_specs=[
            pl.BlockSpec((1, 1, TILE_R), lambda b, t: (b * T + t, 0, 0)),
            pl.BlockSpec((1, 1, TILE_R), lambda b, t: (b * T + t, 0, 0)),
            pl.BlockSpec((1, 1, TILE_R), lambda b, t: (b * T + t, 0, 0)),
            pl.BlockSpec((1, TILE_R, H, W), lambda b, t: (b, t, 0, 0)),
        ],
        out_specs=pl.BlockSpec((1, 1), lambda b, t: (0, 0)),
        out_shape=jax.ShapeDtypeStruct((1, 1), jnp.float32),
        scratch_shapes=[
            pltpu.SMEM((1,), jnp.float32),
            pltpu.SMEM((1,), jnp.float32),
        ],
    )(row, col, sb3, response_map)
    return out[0, 0]
